# baseline (speedup 1.0000x reference)
import functools

import jax
import jax.numpy as jnp
from jax import lax
from jax.experimental import pallas as pl
from jax.experimental.pallas import tpu as pltpu
from jax.experimental.pallas import tpu_sc as plsc


def _make_gather(B, D, C):
    info = plsc.get_sparse_core_info()
    NC, NS = info.num_cores, info.num_subcores
    NW = NC * NS
    assert B % (NW * C) == 0
    b_per_w = B // NW
    n_chunks = b_per_w // C
    mesh = plsc.VectorSubcoreMesh(core_axis_name="c", subcore_axis_name="s")

    @functools.partial(
        pl.kernel,
        out_type=jax.ShapeDtypeStruct((B, D), jnp.float32),
        mesh=mesh,
        scratch_types=[
            pltpu.VMEM((C,), jnp.int32),
            pltpu.VMEM((C, D), jnp.float32),
            pltpu.SemaphoreType.DMA,
        ],
        compiler_params=pltpu.CompilerParams(use_tc_tiling_on_sc=False),
    )
    def gather(idx_hbm, table_hbm, out_hbm, idx_v, rows_v, sem):
        wid = lax.axis_index("s") * NC + lax.axis_index("c")
        base0 = wid * b_per_w

        def chunk(i, carry):
            base = base0 + i * C
            pltpu.sync_copy(idx_hbm.at[pl.ds(base, C)], idx_v)
            pltpu.async_copy(table_hbm.at[idx_v], rows_v, sem).wait()
            pltpu.sync_copy(rows_v, out_hbm.at[pl.ds(base, C)])
            return carry

        lax.fori_loop(0, n_chunks, chunk, 0)

    return gather


def kernel(indices, wte):
    n, s = indices.shape
    _, D = wte.shape
    B = n * s
    gather = _make_gather(B, D, C=512)
    out = gather(indices.reshape(-1), wte)
    return out.reshape(n, s, D)


# speedup vs baseline: 1.0419x; 1.0419x over previous
import functools

import jax
import jax.numpy as jnp
from jax import lax
from jax.experimental import pallas as pl
from jax.experimental.pallas import tpu as pltpu
from jax.experimental.pallas import tpu_sc as plsc


def _make_gather(B, D, C, NB):
    info = plsc.get_sparse_core_info()
    NC, NS = info.num_cores, info.num_subcores
    NW = NC * NS
    b_per_w = B // NW
    n_chunks = b_per_w // C
    n_groups = n_chunks // NB
    assert B % NW == 0 and b_per_w % C == 0 and n_chunks % NB == 0

    mesh = plsc.VectorSubcoreMesh(core_axis_name="c", subcore_axis_name="s")

    @functools.partial(
        pl.kernel,
        out_type=jax.ShapeDtypeStruct((B, D), jnp.float32),
        mesh=mesh,
        scratch_types=[
            pltpu.VMEM((n_chunks, C), jnp.int32),
            pltpu.VMEM((NB, C, D), jnp.float32),
            pltpu.SemaphoreType.DMA((NB,)),
            pltpu.SemaphoreType.DMA((NB,)),
        ],
        compiler_params=pltpu.CompilerParams(use_tc_tiling_on_sc=False),
    )
    def gather(idx_hbm, table_hbm, out_hbm, idx_v, rows_v, gsem, wsem):
        wid = lax.axis_index("s") * NC + lax.axis_index("c")
        base0 = wid * b_per_w
        pltpu.sync_copy(idx_hbm.at[wid], idx_v)

        def gather_copy(i, b):
            return pltpu.make_async_copy(
                table_hbm.at[idx_v.at[i]], rows_v.at[b], gsem.at[b]
            )

        def wb_copy(i, b):
            return pltpu.make_async_copy(
                rows_v.at[b], out_hbm.at[pl.ds(base0 + i * C, C)], wsem.at[b]
            )

        for b in range(NB):
            gather_copy(b, b).start()

        @pl.loop(1, n_groups)
        def _(g):
            i0 = g * NB
            for b in range(NB):
                prev = i0 - NB + b
                gather_copy(prev, b).wait()
                wb_copy(prev, b).start()
            for b in range(NB):
                wb_copy(i0 - NB + b, b).wait()
                gather_copy(i0 + b, b).start()

        last0 = (n_groups - 1) * NB
        for b in range(NB):
            gather_copy(last0 + b, b).wait()
            wb_copy(last0 + b, b).start()
        for b in range(NB):
            wb_copy(last0 + b, b).wait()

    return gather


def kernel(indices, wte):
    n, s = indices.shape
    _, D = wte.shape
    B = n * s
    C, NB = 320, 4
    info = plsc.get_sparse_core_info()
    NW = info.num_cores * info.num_subcores
    gather = _make_gather(B, D, C, NB)
    out = gather(indices.reshape(NW, (B // NW) // C, C), wte)
    return out.reshape(n, s, D)
